# baseline (speedup 1.0000x reference)
import jax
import jax.numpy as jnp
from jax import lax
from jax.experimental import pallas as pl
from jax.experimental.pallas import tpu as pltpu

_B = 64
_C = 2048
_S = 196
_KBLK = 256


def _fused(feat_ref, x0_ref, A_ref, Wl1_ref, bl1_ref, Wr1_ref,
           Wl2_ref, bl2_ref, Wr2_ref, out_ref, x1_ref, agg1_ref):
    k = pl.program_id(0)

    @pl.when(k == 0)
    def _prologue():
        A = A_ref[...]
        rs_col = jnp.sum(A, axis=1, keepdims=True)
        rs_row = jnp.sum(A.T, axis=0, keepdims=True)
        adj = lax.rsqrt(rs_col) * A.T * lax.rsqrt(rs_row)
        maskf = (adj.astype(jnp.int32) != 0).astype(jnp.float32)
        cnt = jnp.maximum(jnp.sum(maskf, axis=0, keepdims=True), 1.0)
        maskn = maskf / cnt
        x0 = x0_ref[...]
        agg0 = lax.dot_general(maskn, x0, (((0,), (0,)), ((), ())),
                               preferred_element_type=jnp.float32)
        x1 = (jnp.dot(agg0, Wl1_ref[...], preferred_element_type=jnp.float32)
              + bl1_ref[...]
              + jnp.dot(x0, Wr1_ref[...], preferred_element_type=jnp.float32))
        x1 = jnp.where(x1 > 0, x1, 0.2 * x1)
        x1_ref[...] = x1
        agg1_ref[...] = lax.dot_general(maskn, x1, (((0,), (0,)), ((), ())),
                                        preferred_element_type=jnp.float32)
        out_ref[...] = jnp.zeros_like(out_ref)

    x2_blk = (jnp.dot(agg1_ref[...], Wl2_ref[...], preferred_element_type=jnp.float32)
              + bl2_ref[...]
              + jnp.dot(x1_ref[...], Wr2_ref[...], preferred_element_type=jnp.float32))
    fblk = jnp.max(feat_ref[...], axis=-1)
    out_ref[...] += lax.dot_general(fblk, x2_blk, (((1,), (1,)), ((), ())),
                                    preferred_element_type=jnp.float32)


def kernel(feature, inp, A, Wl1, bl1, Wr1, Wl2, bl2, Wr2):
    B, C = feature.shape[0], feature.shape[1]
    N = A.shape[0]
    feat3 = feature.reshape(B, C, _S)
    x0 = inp[0]
    nblk = C // _KBLK
    return pl.pallas_call(
        _fused,
        grid=(nblk,),
        in_specs=[
            pl.BlockSpec((B, _KBLK, _S), lambda k: (0, k, 0)),
            pl.BlockSpec(x0.shape, lambda k: (0, 0)),
            pl.BlockSpec(A.shape, lambda k: (0, 0)),
            pl.BlockSpec(Wl1.shape, lambda k: (0, 0)),
            pl.BlockSpec((1, Wl1.shape[1]), lambda k: (0, 0)),
            pl.BlockSpec(Wr1.shape, lambda k: (0, 0)),
            pl.BlockSpec((Wl2.shape[0], _KBLK), lambda k: (0, k)),
            pl.BlockSpec((1, _KBLK), lambda k: (0, k)),
            pl.BlockSpec((Wl2.shape[0], _KBLK), lambda k: (0, k)),
        ],
        out_specs=pl.BlockSpec((B, N), lambda k: (0, 0)),
        out_shape=jax.ShapeDtypeStruct((B, N), jnp.float32),
        scratch_shapes=[
            pltpu.VMEM((N, Wl1.shape[1]), jnp.float32),
            pltpu.VMEM((N, Wl1.shape[1]), jnp.float32),
        ],
        compiler_params=pltpu.CompilerParams(
            dimension_semantics=("arbitrary",),
        ),
    )(feat3, x0, A, Wl1, bl1.reshape(1, -1), Wr1,
      Wl2, bl2.reshape(1, -1), Wr2)


# speedup vs baseline: 4.0625x; 4.0625x over previous
import jax
import jax.numpy as jnp
from jax import lax
from jax.experimental import pallas as pl
from jax.experimental.pallas import tpu as pltpu

_S = 196
_CBLK = 256


def _fused(feat_ref, x0_ref, A_ref, Wl1_ref, bl1_ref, Wr1_ref,
           Wl2_ref, bl2_ref, Wr2_ref, out_ref, x1_ref, agg1_ref):
    k = pl.program_id(0)

    @pl.when(k == 0)
    def _prologue():
        A = A_ref[...]
        rs_col = jnp.sum(A, axis=1, keepdims=True)
        rs_row = jnp.sum(A.T, axis=0, keepdims=True)
        adj = lax.rsqrt(rs_col) * A.T * lax.rsqrt(rs_row)
        maskf = (adj.astype(jnp.int32) != 0).astype(jnp.float32)
        cnt = jnp.maximum(jnp.sum(maskf, axis=0, keepdims=True), 1.0)
        maskn = maskf / cnt
        x0 = x0_ref[...]
        agg0 = lax.dot_general(maskn, x0, (((0,), (0,)), ((), ())),
                               preferred_element_type=jnp.float32)
        x1 = (jnp.dot(agg0, Wl1_ref[...], preferred_element_type=jnp.float32)
              + bl1_ref[...]
              + jnp.dot(x0, Wr1_ref[...], preferred_element_type=jnp.float32))
        x1 = jnp.where(x1 > 0, x1, 0.2 * x1)
        x1_ref[...] = x1
        agg1_ref[...] = lax.dot_general(maskn, x1, (((0,), (0,)), ((), ())),
                                        preferred_element_type=jnp.float32)
        out_ref[...] = jnp.zeros_like(out_ref)

    x2_blk = (jnp.dot(agg1_ref[...], Wl2_ref[...], preferred_element_type=jnp.float32)
              + bl2_ref[...]
              + jnp.dot(x1_ref[...], Wr2_ref[...], preferred_element_type=jnp.float32))
    fblk = jnp.max(feat_ref[...], axis=0)
    out_ref[...] += lax.dot_general(fblk, x2_blk, (((1,), (1,)), ((), ())),
                                    preferred_element_type=jnp.float32)


def kernel(feature, inp, A, Wl1, bl1, Wr1, Wl2, bl2, Wr2):
    B, C = feature.shape[0], feature.shape[1]
    N = A.shape[0]
    featT = jnp.transpose(feature, (2, 3, 0, 1)).reshape(_S, B, C)
    x0 = inp[0]
    nblk = C // _CBLK
    return pl.pallas_call(
        _fused,
        grid=(nblk,),
        in_specs=[
            pl.BlockSpec((_S, B, _CBLK), lambda k: (0, 0, k)),
            pl.BlockSpec(x0.shape, lambda k: (0, 0)),
            pl.BlockSpec(A.shape, lambda k: (0, 0)),
            pl.BlockSpec(Wl1.shape, lambda k: (0, 0)),
            pl.BlockSpec((1, Wl1.shape[1]), lambda k: (0, 0)),
            pl.BlockSpec(Wr1.shape, lambda k: (0, 0)),
            pl.BlockSpec((Wl2.shape[0], _CBLK), lambda k: (0, k)),
            pl.BlockSpec((1, _CBLK), lambda k: (0, k)),
            pl.BlockSpec((Wl2.shape[0], _CBLK), lambda k: (0, k)),
        ],
        out_specs=pl.BlockSpec((B, N), lambda k: (0, 0)),
        out_shape=jax.ShapeDtypeStruct((B, N), jnp.float32),
        scratch_shapes=[
            pltpu.VMEM((N, Wl1.shape[1]), jnp.float32),
            pltpu.VMEM((N, Wl1.shape[1]), jnp.float32),
        ],
        compiler_params=pltpu.CompilerParams(
            dimension_semantics=("arbitrary",),
        ),
    )(featT, x0, A, Wl1, bl1.reshape(1, -1), Wr1,
      Wl2, bl2.reshape(1, -1), Wr2)


# speedup vs baseline: 4.1162x; 1.0132x over previous
import jax
import jax.numpy as jnp
from jax import lax
from jax.experimental import pallas as pl
from jax.experimental.pallas import tpu as pltpu

_S = 196
_NS = 16
_SCHUNK = 14
_WBLK = 128


def _fused(feat_ref, x0_ref, A_ref, Wl1_ref, bl1_ref, Wr1_ref,
           Wl2_ref, bl2_ref, Wr2_ref, out_ref, x1_ref, agg1_ref,
           facc_ref, x2_ref):
    i = pl.program_id(0)

    @pl.when(i == 0)
    def _prologue():
        A = A_ref[...]
        rs_col = jnp.sum(A, axis=1, keepdims=True)
        rs_row = jnp.sum(A.T, axis=0, keepdims=True)
        adj = lax.rsqrt(rs_col) * A.T * lax.rsqrt(rs_row)
        maskf = (adj.astype(jnp.int32) != 0).astype(jnp.float32)
        cnt = jnp.maximum(jnp.sum(maskf, axis=0, keepdims=True), 1.0)
        maskn = maskf / cnt
        x0 = x0_ref[...]
        agg0 = lax.dot_general(maskn, x0, (((0,), (0,)), ((), ())),
                               preferred_element_type=jnp.float32)
        x1 = (jnp.dot(agg0, Wl1_ref[...], preferred_element_type=jnp.float32)
              + bl1_ref[...]
              + jnp.dot(x0, Wr1_ref[...], preferred_element_type=jnp.float32))
        x1 = jnp.where(x1 > 0, x1, 0.2 * x1)
        x1_ref[...] = x1
        agg1_ref[...] = lax.dot_general(maskn, x1, (((0,), (0,)), ((), ())),
                                        preferred_element_type=jnp.float32)

    slab_max = jnp.max(feat_ref[...], axis=0)

    @pl.when(i == 0)
    def _init_max():
        facc_ref[...] = slab_max

    @pl.when(i > 0)
    def _fold_max():
        facc_ref[...] = jnp.maximum(facc_ref[...], slab_max)

    x2_ref[:, pl.ds(i * _WBLK, _WBLK)] = (
        jnp.dot(agg1_ref[...], Wl2_ref[...], preferred_element_type=jnp.float32)
        + bl2_ref[...]
        + jnp.dot(x1_ref[...], Wr2_ref[...], preferred_element_type=jnp.float32))

    @pl.when(i == _NS - 1)
    def _classify():
        out_ref[...] = lax.dot_general(
            facc_ref[...], x2_ref[...], (((1,), (1,)), ((), ())),
            preferred_element_type=jnp.float32)


def kernel(feature, inp, A, Wl1, bl1, Wr1, Wl2, bl2, Wr2):
    B, C = feature.shape[0], feature.shape[1]
    N = A.shape[0]
    H1 = Wl1.shape[1]
    featT = jnp.transpose(feature, (2, 3, 0, 1)).reshape(_S, B, C)
    x0 = inp[0]
    nslab = _S // _SCHUNK
    return pl.pallas_call(
        _fused,
        grid=(_NS,),
        in_specs=[
            pl.BlockSpec((_SCHUNK, B, C), lambda i: (jnp.minimum(i, nslab - 1), 0, 0)),
            pl.BlockSpec(x0.shape, lambda i: (0, 0)),
            pl.BlockSpec(A.shape, lambda i: (0, 0)),
            pl.BlockSpec(Wl1.shape, lambda i: (0, 0)),
            pl.BlockSpec((1, H1), lambda i: (0, 0)),
            pl.BlockSpec(Wr1.shape, lambda i: (0, 0)),
            pl.BlockSpec((Wl2.shape[0], _WBLK), lambda i: (0, i)),
            pl.BlockSpec((1, _WBLK), lambda i: (0, i)),
            pl.BlockSpec((Wl2.shape[0], _WBLK), lambda i: (0, i)),
        ],
        out_specs=pl.BlockSpec((B, N), lambda i: (0, 0)),
        out_shape=jax.ShapeDtypeStruct((B, N), jnp.float32),
        scratch_shapes=[
            pltpu.VMEM((N, H1), jnp.float32),
            pltpu.VMEM((N, H1), jnp.float32),
            pltpu.VMEM((B, C), jnp.float32),
            pltpu.VMEM((N, C), jnp.float32),
        ],
        compiler_params=pltpu.CompilerParams(
            dimension_semantics=("arbitrary",),
        ),
    )(featT, x0, A, Wl1, bl1.reshape(1, -1), Wr1,
      Wl2, bl2.reshape(1, -1), Wr2)
